# Initial kernel scaffold; baseline (speedup 1.0000x reference)
import jax
import jax.numpy as jnp
from jax.experimental import pallas as pl


def kernel(x, W1, b1, W2, b2, W3, b3, Wc, D1, c1, D2, c2, D3, c3):
    raise NotImplementedError("write your pallas kernel here")


# speedup vs baseline: 1.9099x; 1.9099x over previous
import functools

import jax
import jax.numpy as jnp
from jax.experimental import pallas as pl


def _fused_body(x_ref, w1_ref, b1_ref, w2_ref, b2_ref, w3_ref, b3_ref,
                wc_ref, d1_ref, c1_ref, d2_ref, c2_ref, d3_ref, c3_ref,
                xr_ref, ze_ref, emb_ref):
    xb = x_ref[...]
    h = jnp.dot(xb, w1_ref[...], preferred_element_type=jnp.float32) + b1_ref[...]
    h = jnp.maximum(h, 0.0)
    h = jnp.dot(h, w2_ref[...], preferred_element_type=jnp.float32) + b2_ref[...]
    h = jnp.maximum(h, 0.0)
    ze = jnp.dot(h, w3_ref[...], preferred_element_type=jnp.float32) + b3_ref[...]
    ze_ref[...] = ze

    wc = wc_ref[...]
    cnorm = jnp.sum(wc * wc, axis=0, keepdims=True)
    dist = cnorm - 2.0 * jnp.dot(ze, wc, preferred_element_type=jnp.float32)
    idx = jnp.argmin(dist, axis=1)
    onehot = (jax.lax.broadcasted_iota(jnp.int32, dist.shape, 1)
              == idx[:, None]).astype(jnp.float32)
    emb = jax.lax.dot_general(onehot, wc, (((1,), (1,)), ((), ())),
                              preferred_element_type=jnp.float32)
    emb_ref[...] = emb

    h = jnp.dot(emb, d1_ref[...], preferred_element_type=jnp.float32) + c1_ref[...]
    h = jnp.maximum(h, 0.0)
    h = jnp.dot(h, d2_ref[...], preferred_element_type=jnp.float32) + c2_ref[...]
    h = jnp.maximum(h, 0.0)
    xr_ref[...] = (jnp.dot(h, d3_ref[...], preferred_element_type=jnp.float32)
                   + c3_ref[...])


@functools.partial(jax.jit, static_argnames=("bb",))
def _run(x, W1, b1, W2, b2, W3, b3, Wc, D1, c1, D2, c2, D3, c3, bb=1024):
    B, IN = x.shape
    HID = W1.shape[1]
    HALF = W2.shape[1]
    EMB = W3.shape[1]
    K = Wc.shape[1]
    grid = (B // bb,)

    def full(a):
        return pl.BlockSpec(a.shape, lambda i: (0,) * a.ndim)

    b1r, b2r, b3r = b1[None, :], b2[None, :], b3[None, :]
    c1r, c2r, c3r = c1[None, :], c2[None, :], c3[None, :]

    batch_spec = pl.BlockSpec((bb, IN), lambda i: (i, 0))
    out_shapes = (
        jax.ShapeDtypeStruct((B, IN), jnp.float32),
        jax.ShapeDtypeStruct((B, EMB), jnp.float32),
        jax.ShapeDtypeStruct((B, EMB), jnp.float32),
    )
    out_specs = (
        pl.BlockSpec((bb, IN), lambda i: (i, 0)),
        pl.BlockSpec((bb, EMB), lambda i: (i, 0)),
        pl.BlockSpec((bb, EMB), lambda i: (i, 0)),
    )
    in_specs = [batch_spec] + [full(a) for a in
                               (W1, b1r, W2, b2r, W3, b3r, Wc,
                                D1, c1r, D2, c2r, D3, c3r)]
    return pl.pallas_call(
        _fused_body,
        grid=grid,
        in_specs=in_specs,
        out_specs=out_specs,
        out_shape=out_shapes,
    )(x, W1, b1r, W2, b2r, W3, b3r, Wc, D1, c1r, D2, c2r, D3, c3r)


def kernel(x, W1, b1, W2, b2, W3, b3, Wc, D1, c1, D2, c2, D3, c3):
    x_recon, z_e, emb = _run(x, W1, b1, W2, b2, W3, b3, Wc,
                             D1, c1, D2, c2, D3, c3)
    return (x_recon, z_e, emb)


# speedup vs baseline: 1.9115x; 1.0008x over previous
import functools

import jax
import jax.numpy as jnp
from jax.experimental import pallas as pl
from jax.experimental.pallas import tpu as pltpu


def _fused_body(x_ref, w1_ref, b1_ref, w2_ref, b2_ref, w3_ref, b3_ref,
                wc_ref, d1_ref, c1_ref, d2_ref, c2_ref, d3_ref, c3_ref,
                xr_ref, ze_ref, emb_ref):
    xb = x_ref[...]
    h = jnp.dot(xb, w1_ref[...], preferred_element_type=jnp.float32) + b1_ref[...]
    h = jnp.maximum(h, 0.0)
    h = jnp.dot(h, w2_ref[...], preferred_element_type=jnp.float32) + b2_ref[...]
    h = jnp.maximum(h, 0.0)
    ze = jnp.dot(h, w3_ref[...], preferred_element_type=jnp.float32) + b3_ref[...]
    ze_ref[...] = ze

    wc = wc_ref[...]
    cnorm = jnp.sum(wc * wc, axis=0, keepdims=True)
    dist = cnorm - 2.0 * jnp.dot(ze, wc, preferred_element_type=jnp.float32)
    idx = jnp.argmin(dist, axis=1)
    onehot = (jax.lax.broadcasted_iota(jnp.int32, dist.shape, 1)
              == idx[:, None]).astype(jnp.float32)
    emb = jax.lax.dot_general(onehot, wc, (((1,), (1,)), ((), ())),
                              preferred_element_type=jnp.float32)
    emb_ref[...] = emb

    h = jnp.dot(emb, d1_ref[...], preferred_element_type=jnp.float32) + c1_ref[...]
    h = jnp.maximum(h, 0.0)
    h = jnp.dot(h, d2_ref[...], preferred_element_type=jnp.float32) + c2_ref[...]
    h = jnp.maximum(h, 0.0)
    xr_ref[...] = (jnp.dot(h, d3_ref[...], preferred_element_type=jnp.float32)
                   + c3_ref[...])


@functools.partial(jax.jit, static_argnames=("bb",))
def _run(x, W1, b1, W2, b2, W3, b3, Wc, D1, c1, D2, c2, D3, c3, bb=1024):
    B, IN = x.shape
    HID = W1.shape[1]
    HALF = W2.shape[1]
    EMB = W3.shape[1]
    K = Wc.shape[1]
    grid = (B // bb,)

    def full(a):
        return pl.BlockSpec(a.shape, lambda i: (0,) * a.ndim)

    b1r, b2r, b3r = b1[None, :], b2[None, :], b3[None, :]
    c1r, c2r, c3r = c1[None, :], c2[None, :], c3[None, :]

    batch_spec = pl.BlockSpec((bb, IN), lambda i: (i, 0))
    out_shapes = (
        jax.ShapeDtypeStruct((B, IN), jnp.float32),
        jax.ShapeDtypeStruct((B, EMB), jnp.float32),
        jax.ShapeDtypeStruct((B, EMB), jnp.float32),
    )
    out_specs = (
        pl.BlockSpec((bb, IN), lambda i: (i, 0)),
        pl.BlockSpec((bb, EMB), lambda i: (i, 0)),
        pl.BlockSpec((bb, EMB), lambda i: (i, 0)),
    )
    in_specs = [batch_spec] + [full(a) for a in
                               (W1, b1r, W2, b2r, W3, b3r, Wc,
                                D1, c1r, D2, c2r, D3, c3r)]
    return pl.pallas_call(
        _fused_body,
        grid=grid,
        in_specs=in_specs,
        out_specs=out_specs,
        out_shape=out_shapes,
        compiler_params=pltpu.CompilerParams(
            dimension_semantics=("parallel",)),
    )(x, W1, b1r, W2, b2r, W3, b3r, Wc, D1, c1r, D2, c2r, D3, c3r)


def kernel(x, W1, b1, W2, b2, W3, b3, Wc, D1, c1, D2, c2, D3, c3):
    x_recon, z_e, emb = _run(x, W1, b1, W2, b2, W3, b3, Wc,
                             D1, c1, D2, c2, D3, c3)
    return (x_recon, z_e, emb)


# speedup vs baseline: 1.9818x; 1.0368x over previous
import functools

import jax
import jax.numpy as jnp
from jax.experimental import pallas as pl
from jax.experimental.pallas import tpu as pltpu


def _fused_body(x_ref, w1_ref, b1_ref, w2_ref, b2_ref, w3_ref, b3_ref,
                wc_ref, d1_ref, c1_ref, d2_ref, c2_ref, d3_ref, c3_ref,
                xr_ref, ze_ref, emb_ref):
    xb = x_ref[...]
    h = jnp.dot(xb, w1_ref[...], preferred_element_type=jnp.float32) + b1_ref[...]
    h = jnp.maximum(h, 0.0)
    h = jnp.dot(h, w2_ref[...], preferred_element_type=jnp.float32) + b2_ref[...]
    h = jnp.maximum(h, 0.0)
    ze = jnp.dot(h, w3_ref[...], preferred_element_type=jnp.float32) + b3_ref[...]
    ze_ref[...] = ze

    wc = wc_ref[...]
    cnorm = jnp.sum(wc * wc, axis=0, keepdims=True)
    dist = cnorm - 2.0 * jnp.dot(ze, wc, preferred_element_type=jnp.float32)
    idx = jnp.argmin(dist, axis=1)
    onehot = (jax.lax.broadcasted_iota(jnp.int32, dist.shape, 1)
              == idx[:, None]).astype(jnp.float32)
    emb = jax.lax.dot_general(onehot, wc, (((1,), (1,)), ((), ())),
                              preferred_element_type=jnp.float32)
    emb_ref[...] = emb

    h = jnp.dot(emb, d1_ref[...], preferred_element_type=jnp.float32) + c1_ref[...]
    h = jnp.maximum(h, 0.0)
    h = jnp.dot(h, d2_ref[...], preferred_element_type=jnp.float32) + c2_ref[...]
    h = jnp.maximum(h, 0.0)
    xr_ref[...] = (jnp.dot(h, d3_ref[...], preferred_element_type=jnp.float32)
                   + c3_ref[...])


@functools.partial(jax.jit, static_argnames=("bb",))
def _run(x, W1, b1, W2, b2, W3, b3, Wc, D1, c1, D2, c2, D3, c3, bb=2048):
    B, IN = x.shape
    HID = W1.shape[1]
    HALF = W2.shape[1]
    EMB = W3.shape[1]
    K = Wc.shape[1]
    grid = (B // bb,)

    def full(a):
        return pl.BlockSpec(a.shape, lambda i: (0,) * a.ndim)

    b1r, b2r, b3r = b1[None, :], b2[None, :], b3[None, :]
    c1r, c2r, c3r = c1[None, :], c2[None, :], c3[None, :]

    batch_spec = pl.BlockSpec((bb, IN), lambda i: (i, 0))
    out_shapes = (
        jax.ShapeDtypeStruct((B, IN), jnp.float32),
        jax.ShapeDtypeStruct((B, EMB), jnp.float32),
        jax.ShapeDtypeStruct((B, EMB), jnp.float32),
    )
    out_specs = (
        pl.BlockSpec((bb, IN), lambda i: (i, 0)),
        pl.BlockSpec((bb, EMB), lambda i: (i, 0)),
        pl.BlockSpec((bb, EMB), lambda i: (i, 0)),
    )
    in_specs = [batch_spec] + [full(a) for a in
                               (W1, b1r, W2, b2r, W3, b3r, Wc,
                                D1, c1r, D2, c2r, D3, c3r)]
    return pl.pallas_call(
        _fused_body,
        grid=grid,
        in_specs=in_specs,
        out_specs=out_specs,
        out_shape=out_shapes,
        compiler_params=pltpu.CompilerParams(
            dimension_semantics=("parallel",)),
    )(x, W1, b1r, W2, b2r, W3, b3r, Wc, D1, c1r, D2, c2r, D3, c3r)


def kernel(x, W1, b1, W2, b2, W3, b3, Wc, D1, c1, D2, c2, D3, c3):
    x_recon, z_e, emb = _run(x, W1, b1, W2, b2, W3, b3, Wc,
                             D1, c1, D2, c2, D3, c3)
    return (x_recon, z_e, emb)
